# baseline (speedup 1.0000x reference)
import functools

import jax
import jax.numpy as jnp
from jax import lax
from jax.experimental import pallas as pl
from jax.experimental.pallas import tpu as pltpu
from jax.experimental.pallas import tpu_sc as plsc

_NC = 2
_NS = 16
_NW = _NC * _NS
_LANES = 16
_C = 16384


def _sc_partial_sums(n_total):
  assert n_total % (_NW * _C) == 0
  per_worker = n_total // _NW
  num_chunks = per_worker // _C
  mesh = plsc.VectorSubcoreMesh(core_axis_name="c", subcore_axis_name="s")

  @functools.partial(
      pl.kernel,
      out_type=jax.ShapeDtypeStruct((_NW, 2 * _LANES), jnp.float32),
      mesh=mesh,
      scratch_types=[
          pltpu.VMEM((2, _C), jnp.float32),
          pltpu.VMEM((2, _C), jnp.float32),
          pltpu.VMEM((2, _C), jnp.int32),
          pltpu.VMEM((2 * _LANES,), jnp.float32),
          pltpu.SemaphoreType.DMA,
          pltpu.SemaphoreType.DMA,
      ],
  )
  def k(pred_hbm, gt_hbm, mask_hbm, out_hbm, pbuf, gbuf, mbuf, obuf, sem0,
        sem1):
    wid = lax.axis_index("s") * _NC + lax.axis_index("c")
    base = wid * per_worker
    sems = (sem0, sem1)

    def fire(j, b):
      off = base + j * _C
      return [
          pltpu.async_copy(pred_hbm.at[pl.ds(off, _C)], pbuf.at[b], sems[b]),
          pltpu.async_copy(gt_hbm.at[pl.ds(off, _C)], gbuf.at[b], sems[b]),
          pltpu.async_copy(mask_hbm.at[pl.ds(off, _C)], mbuf.at[b], sems[b]),
      ]

    acc_s = jnp.zeros((_LANES,), jnp.float32)
    acc_c = jnp.zeros((_LANES,), jnp.float32)
    handles = [fire(0, 0), None]
    for j in range(num_chunks):
      b = j & 1
      if j + 1 < num_chunks:
        handles[1 - b] = fire(j + 1, 1 - b)
      for h in handles[b]:
        h.wait()

      def body(i, carry, b=b):
        s, c = carry
        off = i * _LANES
        p = pbuf[b, pl.ds(off, _LANES)]
        g = gbuf[b, pl.ds(off, _LANES)]
        m = mbuf[b, pl.ds(off, _LANES)]
        sel = m > 0
        s = s + jnp.where(sel, jnp.abs(p - g), 0.0)
        c = c + jnp.where(sel, 1.0, 0.0)
        return (s, c)

      acc_s, acc_c = lax.fori_loop(0, _C // _LANES, body, (acc_s, acc_c),
                                   unroll=8)

    obuf[pl.ds(0, _LANES)] = acc_s
    obuf[pl.ds(_LANES, _LANES)] = acc_c
    pltpu.sync_copy(obuf, out_hbm.at[wid])

  return k


def kernel(predictions, gt_dose, possible_dose_mask):
  n = predictions.size
  p = predictions.reshape(n)
  g = gt_dose.reshape(n)
  m = possible_dose_mask.reshape(n)
  parts = _sc_partial_sums(n)(p, g, m)
  total = jnp.sum(parts[:, :_LANES])
  count = jnp.sum(parts[:, _LANES:])
  return total / jnp.maximum(count, 1.0)
